# Initial kernel scaffold; baseline (speedup 1.0000x reference)
import jax
import jax.numpy as jnp
from jax.experimental import pallas as pl


def kernel(timestep, pe):
    raise NotImplementedError("write your pallas kernel here")


# speedup vs baseline: 5.1108x; 5.1108x over previous
import functools

import jax
import jax.numpy as jnp
from jax import lax
from jax.experimental import pallas as pl
from jax.experimental.pallas import tpu as pltpu
from jax.experimental.pallas import tpu_sc as plsc

EMBED = 64
NUM_CORES = 2
NUM_SUBCORES = 16
NUM_WORKERS = NUM_CORES * NUM_SUBCORES
CHUNK = 800


def _make_gather(total, table_rows):
    assert total % (NUM_WORKERS * CHUNK) == 0
    per_worker = total // NUM_WORKERS
    num_chunks = per_worker // CHUNK
    assert num_chunks % 2 == 0

    mesh = plsc.VectorSubcoreMesh(
        core_axis_name="c", subcore_axis_name="s",
        num_cores=NUM_CORES, num_subcores=NUM_SUBCORES)

    @functools.partial(
        pl.kernel,
        out_type=jax.ShapeDtypeStruct((total, EMBED), jnp.float32),
        mesh=mesh,
        compiler_params=pltpu.CompilerParams(use_tc_tiling_on_sc=False),
        scratch_types=[
            pltpu.VMEM((CHUNK,), jnp.int32),
            pltpu.VMEM((CHUNK,), jnp.int32),
            pltpu.VMEM((CHUNK, EMBED), jnp.float32),
            pltpu.VMEM((CHUNK, EMBED), jnp.float32),
            pltpu.SemaphoreType.DMA,
            pltpu.SemaphoreType.DMA,
            pltpu.SemaphoreType.DMA,
            pltpu.SemaphoreType.DMA,
        ],
    )
    def gather_kernel(idx_hbm, pe_hbm, out_hbm,
                      idx0, idx1, rows0, rows1, gsem0, gsem1, osem0, osem1):
        wid = lax.axis_index("s") * NUM_CORES + lax.axis_index("c")
        base = wid * per_worker
        slots = ((idx0, rows0, gsem0, osem0), (idx1, rows1, gsem1, osem1))

        def start_gather(g, idx_v, rows_v, gsem):
            pltpu.sync_copy(idx_hbm.at[pl.ds(base + g * CHUNK, CHUNK)], idx_v)
            pltpu.async_copy(pe_hbm.at[idx_v], rows_v, gsem)

        start_gather(0, idx0, rows0, gsem0)

        @pl.loop(0, num_chunks, step=2)
        def _(g0):
            for k in range(2):
                g = g0 + k
                idx_v, rows_v, gsem, osem = slots[k]
                oidx_v, orows_v, ogsem, oosem = slots[1 - k]

                @pl.when(g + 1 < num_chunks)
                def _():
                    @pl.when(g >= 1)
                    def _():
                        pltpu.make_async_copy(
                            orows_v, out_hbm.at[pl.ds(0, CHUNK)], oosem).wait()
                    start_gather(g + 1, oidx_v, orows_v, ogsem)

                pltpu.make_async_copy(pe_hbm.at[idx_v], rows_v, gsem).wait()
                pltpu.async_copy(
                    rows_v, out_hbm.at[pl.ds(base + g * CHUNK, CHUNK)], osem)

        pltpu.make_async_copy(rows0, out_hbm.at[pl.ds(0, CHUNK)], osem0).wait()
        pltpu.make_async_copy(rows1, out_hbm.at[pl.ds(0, CHUNK)], osem1).wait()

    return gather_kernel


def kernel(timestep, pe):
    lead_shape = timestep.shape
    idx = timestep.reshape(-1)
    out = _make_gather(idx.shape[0], pe.shape[0])(idx, pe)
    return out.reshape(*lead_shape, EMBED)


# speedup vs baseline: 5.1726x; 1.0121x over previous
import functools

import jax
import jax.numpy as jnp
from jax import lax
from jax.experimental import pallas as pl
from jax.experimental.pallas import tpu as pltpu
from jax.experimental.pallas import tpu_sc as plsc

EMBED = 64
NUM_CORES = 2
NUM_SUBCORES = 16
NUM_WORKERS = NUM_CORES * NUM_SUBCORES
CHUNK = 400
NBUF = 4


def _make_gather(total):
    assert total % (NUM_WORKERS * CHUNK) == 0
    per_worker = total // NUM_WORKERS
    num_chunks = per_worker // CHUNK
    assert num_chunks % NBUF == 0 and num_chunks > 2 * NBUF

    mesh = plsc.VectorSubcoreMesh(
        core_axis_name="c", subcore_axis_name="s",
        num_cores=NUM_CORES, num_subcores=NUM_SUBCORES)

    @functools.partial(
        pl.kernel,
        out_type=jax.ShapeDtypeStruct((total, EMBED), jnp.float32),
        mesh=mesh,
        compiler_params=pltpu.CompilerParams(use_tc_tiling_on_sc=False),
        scratch_types=[
            [pltpu.VMEM((CHUNK,), jnp.int32) for _ in range(NBUF)],
            [pltpu.VMEM((CHUNK, EMBED), jnp.float32) for _ in range(NBUF)],
            [pltpu.SemaphoreType.DMA for _ in range(NBUF)],
            [pltpu.SemaphoreType.DMA for _ in range(NBUF)],
            [pltpu.SemaphoreType.DMA for _ in range(NBUF)],
        ],
    )
    def gather_kernel(idx_hbm, pe_hbm, out_hbm, idxs, rows, isems, gsems, osems):
        wid = lax.axis_index("s") * NUM_CORES + lax.axis_index("c")
        base = wid * per_worker

        def start_idx(g, s):
            pltpu.async_copy(
                idx_hbm.at[pl.ds(base + g * CHUNK, CHUNK)], idxs[s], isems[s])

        def wait_idx(s):
            pltpu.make_async_copy(
                idx_hbm.at[pl.ds(0, CHUNK)], idxs[s], isems[s]).wait()

        def start_gather(s):
            pltpu.async_copy(pe_hbm.at[idxs[s]], rows[s], gsems[s])

        def wait_gather(s):
            pltpu.make_async_copy(pe_hbm.at[idxs[s]], rows[s], gsems[s]).wait()

        def start_out(g, s):
            pltpu.async_copy(
                rows[s], out_hbm.at[pl.ds(base + g * CHUNK, CHUNK)], osems[s])

        def wait_out(s):
            pltpu.make_async_copy(
                rows[s], out_hbm.at[pl.ds(0, CHUNK)], osems[s]).wait()

        for s in range(NBUF):
            start_idx(s, s)
        for s in range(NBUF - 1):
            wait_idx(s)
            start_gather(s)

        @pl.loop(0, num_chunks, step=NBUF)
        def _(g0):
            for k in range(NBUF):
                s = k
                t = (k + NBUF - 1) % NBUF
                g = g0 + k

                @pl.when(g + NBUF - 1 < num_chunks)
                def _():
                    @pl.when(g >= 1)
                    def _():
                        wait_out(t)
                    wait_idx(t)
                    start_gather(t)

                wait_gather(s)
                start_out(g, s)

                @pl.when(g + NBUF < num_chunks)
                def _():
                    start_idx(g + NBUF, s)

        for s in range(NBUF):
            wait_out(s)

    return gather_kernel


def kernel(timestep, pe):
    lead_shape = timestep.shape
    idx = timestep.reshape(-1)
    out = _make_gather(idx.shape[0])(idx, pe)
    return out.reshape(*lead_shape, EMBED)


# speedup vs baseline: 5.1743x; 1.0003x over previous
import functools

import jax
import jax.numpy as jnp
from jax import lax
from jax.experimental import pallas as pl
from jax.experimental.pallas import tpu as pltpu
from jax.experimental.pallas import tpu_sc as plsc

EMBED = 64
NUM_CORES = 2
NUM_SUBCORES = 16
NUM_WORKERS = NUM_CORES * NUM_SUBCORES
NBUF = 4


def _make_gather(n_seq, seq_len):
    total = n_seq * seq_len
    chunk = seq_len
    assert total % (NUM_WORKERS * chunk) == 0 and chunk % 8 == 0
    per_worker = total // NUM_WORKERS
    num_chunks = per_worker // chunk
    assert num_chunks % NBUF == 0 and num_chunks > 2 * NBUF

    mesh = plsc.VectorSubcoreMesh(
        core_axis_name="c", subcore_axis_name="s",
        num_cores=NUM_CORES, num_subcores=NUM_SUBCORES)

    @functools.partial(
        pl.kernel,
        out_type=jax.ShapeDtypeStruct((n_seq, seq_len, EMBED), jnp.float32),
        mesh=mesh,
        compiler_params=pltpu.CompilerParams(use_tc_tiling_on_sc=False),
        scratch_types=[
            [pltpu.VMEM((chunk,), jnp.int32) for _ in range(NBUF)],
            [pltpu.VMEM((chunk, EMBED), jnp.float32) for _ in range(NBUF)],
            [pltpu.SemaphoreType.DMA for _ in range(NBUF)],
            [pltpu.SemaphoreType.DMA for _ in range(NBUF)],
            [pltpu.SemaphoreType.DMA for _ in range(NBUF)],
        ],
    )
    def gather_kernel(idx_hbm, pe_hbm, out_hbm, idxs, rows, isems, gsems, osems):
        wid = lax.axis_index("s") * NUM_CORES + lax.axis_index("c")
        seq0 = wid * num_chunks
        base = wid * per_worker

        def start_idx(g, s):
            pltpu.async_copy(
                idx_hbm.at[pl.ds(base + g * chunk, chunk)], idxs[s], isems[s])

        def wait_idx(s):
            pltpu.make_async_copy(
                idx_hbm.at[pl.ds(0, chunk)], idxs[s], isems[s]).wait()

        def start_gather(s):
            pltpu.async_copy(pe_hbm.at[idxs[s]], rows[s], gsems[s])

        def wait_gather(s):
            pltpu.make_async_copy(pe_hbm.at[idxs[s]], rows[s], gsems[s]).wait()

        def start_out(g, s):
            pltpu.async_copy(rows[s], out_hbm.at[seq0 + g], osems[s])

        def wait_out(s):
            pltpu.make_async_copy(rows[s], out_hbm.at[0], osems[s]).wait()

        for s in range(NBUF):
            start_idx(s, s)
        for s in range(NBUF - 1):
            wait_idx(s)
            start_gather(s)

        @pl.loop(0, num_chunks, step=NBUF)
        def _(g0):
            for k in range(NBUF):
                s = k
                t = (k + NBUF - 1) % NBUF
                g = g0 + k

                @pl.when(g + NBUF - 1 < num_chunks)
                def _():
                    @pl.when(g >= 1)
                    def _():
                        wait_out(t)
                    wait_idx(t)
                    start_gather(t)

                wait_gather(s)
                start_out(g, s)

                @pl.when(g + NBUF < num_chunks)
                def _():
                    start_idx(g + NBUF, s)

        for s in range(NBUF):
            wait_out(s)

    return gather_kernel


def kernel(timestep, pe):
    n_seq, seq_len = timestep.shape
    idx = timestep.reshape(-1)
    return _make_gather(n_seq, seq_len)(idx, pe)
